# Initial kernel scaffold; baseline (speedup 1.0000x reference)
import functools

import jax
import jax.numpy as jnp
from jax import lax
from jax.experimental import pallas as pl
from jax.experimental.pallas import tpu as pltpu
from jax.experimental.pallas import tpu_sc as plsc

_HEADS = 4
_D = 32
_F = 64
_ACC_W = 80


def _tc_project(feat, W, n):
    nb = 2000
    grid = (2, n // nb)

    def body(f_ref, w_ref, o_ref):
        o_ref[...] = jnp.dot(f_ref[...], w_ref[...],
                             preferred_element_type=jnp.float32)

    return pl.pallas_call(
        body,
        grid=grid,
        in_specs=[
            pl.BlockSpec((nb, feat.shape[1]), lambda p, i: (i, 0)),
            pl.BlockSpec((feat.shape[1], _F), lambda p, i: (0, p)),
        ],
        out_specs=pl.BlockSpec((nb, _F), lambda p, i: (p * (n // nb) + i, 0)),
        out_shape=jax.ShapeDtypeStruct((2 * n, _F), jnp.float32),
    )(feat, W)


def _sc_gat(h2, src, dst, n, e):
    num_tiles = 16
    per_tile_e = e // num_tiles
    C = 80
    n_chunks = per_tile_e // C
    rows_per_tile = n // num_tiles
    RB = 125
    row_blocks = rows_per_tile // RB
    inv_sqrt_d = float(1.0 / (_D ** 0.5))

    mesh = plsc.VectorSubcoreMesh(core_axis_name="c", subcore_axis_name="s")

    @functools.partial(
        pl.kernel,
        out_type=jax.ShapeDtypeStruct((2 * n, _F), jnp.float32),
        mesh=mesh,
        scratch_types=[
            pltpu.VMEM((C,), jnp.int32),
            pltpu.VMEM((C,), jnp.int32),
            pltpu.VMEM((C,), jnp.int32),
            pltpu.VMEM((C, _F), jnp.float32),
            pltpu.VMEM((C, _F), jnp.float32),
            pltpu.VMEM((C, _ACC_W), jnp.float32),
            pltpu.VMEM((RB, _ACC_W), jnp.float32),
            pltpu.VMEM((RB, _F), jnp.float32),
            pltpu.VMEM_SHARED((n, _ACC_W), jnp.float32),
        ],
    )
    def k(h_hbm, src_hbm, dst_hbm, out_hbm,
          sidx, didx, gdidx, srows, drows, scaled, zbuf, obuf, acc):
        cid = lax.axis_index("c")
        sid = lax.axis_index("s")
        zero16 = jnp.zeros((16,), jnp.float32)

        @pl.loop(0, RB)
        def _(r):
            for j in range(_ACC_W // 16):
                zbuf[r, pl.ds(16 * j, 16)] = zero16

        row0 = sid * rows_per_tile
        for b in range(row_blocks):
            pltpu.sync_copy(zbuf, acc.at[pl.ds(row0 + b * RB, RB)])

        @pl.loop(0, C)
        def _(ei):
            scaled[ei, pl.ds(_F, 16)] = zero16

        plsc.subcore_barrier()

        tbase = cid * n

        @pl.loop(0, n_chunks)
        def _(chunk):
            base_e = sid * per_tile_e + chunk * C
            pltpu.sync_copy(src_hbm.at[pl.ds(base_e, C)], sidx)
            pltpu.sync_copy(dst_hbm.at[pl.ds(base_e, C)], didx)

            @pl.loop(0, C, step=16)
            def _(i):
                sidx[pl.ds(i, 16)] = sidx[pl.ds(i, 16)] + tbase
                gdidx[pl.ds(i, 16)] = didx[pl.ds(i, 16)] + tbase

            pltpu.sync_copy(h_hbm.at[sidx], srows)
            pltpu.sync_copy(h_hbm.at[gdidx], drows)

            @pl.loop(0, C, step=16)
            def _(eb):
                rows = lax.iota(jnp.int32, 16) + eb

                def dot_head(lo):
                    def body(d, a):
                        dv = jnp.full((16,), d, jnp.int32)
                        sc = plsc.load_gather(srows, [rows, dv])
                        dc = plsc.load_gather(drows, [rows, dv])
                        return a + sc * dc
                    return lax.fori_loop(lo, lo + _D, body, zero16,
                                         unroll=8)

                w0 = jnp.exp(dot_head(0) * inv_sqrt_d)
                w1 = jnp.exp(dot_head(_D) * inv_sqrt_d)
                plsc.store_scatter(scaled, [rows, jnp.full((16,), _F, jnp.int32)], w0)
                plsc.store_scatter(scaled, [rows, jnp.full((16,), _F + 1, jnp.int32)], w1)

                def scale_head(lo, w):
                    def body(d, _):
                        dv = jnp.full((16,), d, jnp.int32)
                        sc = plsc.load_gather(srows, [rows, dv])
                        plsc.store_scatter(scaled, [rows, dv], sc * w)
                        return 0
                    lax.fori_loop(lo, lo + _D, body, 0, unroll=8)

                scale_head(0, w0)
                scale_head(_D, w1)

            pltpu.sync_copy(scaled, acc.at[didx], add=True)

        plsc.subcore_barrier()

        for b in range(row_blocks):
            r0 = row0 + b * RB
            pltpu.sync_copy(acc.at[pl.ds(r0, RB)], zbuf)

            @pl.loop(0, RB)
            def _(r):
                i0 = 1.0 / (zbuf[r, _F] + 1e-9)
                i1 = 1.0 / (zbuf[r, _F + 1] + 1e-9)
                for j in range(4):
                    v = zbuf[r, pl.ds(j * 16, 16)]
                    obuf[r, pl.ds(j * 16, 16)] = v * (i0 if j < 2 else i1)

            pltpu.sync_copy(obuf, out_hbm.at[pl.ds(tbase + r0, RB)])

    return k(h2, src, dst)


def kernel(feat, edge_index, W):
    n = feat.shape[0]
    e = edge_index.shape[1]
    h2 = _tc_project(feat, W, n)
    out2 = _sc_gat(h2, edge_index[0], edge_index[1], n, e)
    return out2.reshape(2, n, 2, _D).transpose(1, 0, 2, 3).reshape(n, _HEADS, _D)


# speedup vs baseline: 12.8731x; 12.8731x over previous
import functools

import jax
import jax.numpy as jnp
from jax import lax
from jax.experimental import pallas as pl
from jax.experimental.pallas import tpu as pltpu
from jax.experimental.pallas import tpu_sc as plsc

_HEADS = 4
_D = 32
_F = 64
_ACC_W = 80


def _tc_project(feat, W, n):
    nb = 2000
    k = feat.shape[1]

    def body(f_ref, w_ref, o_ref):
        o_ref[...] = jnp.dot(f_ref[...], w_ref[...],
                             preferred_element_type=jnp.float32)

    return pl.pallas_call(
        body,
        grid=(n // nb,),
        in_specs=[
            pl.BlockSpec((nb, k), lambda i: (i, 0)),
            pl.BlockSpec((k, 2 * _F), lambda i: (0, 0)),
        ],
        out_specs=pl.BlockSpec((nb, 2 * _F), lambda i: (i, 0)),
        out_shape=jax.ShapeDtypeStruct((n, 2 * _F), jnp.float32),
    )(feat, W)


def _sc_gat(h2, src, dst, n, e):
    num_tiles = 16
    per_tile_e = e // num_tiles
    C = 80
    n_chunks = per_tile_e // C
    rows_per_tile = n // num_tiles
    RB = 125
    row_blocks = rows_per_tile // RB
    inv_sqrt_d = float(1.0 / (_D ** 0.5))

    mesh = plsc.VectorSubcoreMesh(core_axis_name="c", subcore_axis_name="s")

    @functools.partial(
        pl.kernel,
        out_type=jax.ShapeDtypeStruct((2 * n, _ACC_W), jnp.float32),
        mesh=mesh,
        compiler_params=pltpu.CompilerParams(use_tc_tiling_on_sc=False,
                                             needs_layout_passes=False),
        scratch_types=[
            pltpu.VMEM((C,), jnp.int32),
            pltpu.VMEM((C,), jnp.int32),
            pltpu.VMEM((C,), jnp.int32),
            pltpu.VMEM((C, _F), jnp.float32),
            pltpu.VMEM((C, _F), jnp.float32),
            pltpu.VMEM((C, _ACC_W), jnp.float32),
            pltpu.VMEM((RB, _ACC_W), jnp.float32),
            pltpu.VMEM_SHARED((n, _ACC_W), jnp.float32),
        ],
    )
    def k(h_hbm, src_hbm, dst_hbm, out_hbm,
          sidx, didx, gdidx, srows, drows, scaled, zbuf, acc):
        cid = lax.axis_index("c")
        sid = lax.axis_index("s")
        zero16 = jnp.zeros((16,), jnp.float32)

        @pl.loop(0, RB)
        def _(r):
            for j in range(_ACC_W // 16):
                zbuf[r, pl.ds(16 * j, 16)] = zero16

        row0 = sid * rows_per_tile
        for b in range(row_blocks):
            pltpu.sync_copy(zbuf, acc.at[pl.ds(row0 + b * RB, RB)])

        @pl.loop(0, C)
        def _(ei):
            scaled[ei, pl.ds(_F, 16)] = zero16

        plsc.subcore_barrier()

        tbase = cid * n

        @pl.loop(0, n_chunks)
        def _(chunk):
            base_e = sid * per_tile_e + chunk * C
            pltpu.sync_copy(src_hbm.at[pl.ds(base_e, C)], sidx)
            pltpu.sync_copy(dst_hbm.at[pl.ds(base_e, C)], didx)

            @pl.loop(0, C, step=16)
            def _(i):
                sidx[pl.ds(i, 16)] = sidx[pl.ds(i, 16)] + tbase
                gdidx[pl.ds(i, 16)] = didx[pl.ds(i, 16)] + tbase

            pltpu.sync_copy(h_hbm.at[sidx], srows)
            pltpu.sync_copy(h_hbm.at[gdidx], drows)

            @pl.loop(0, C, step=16)
            def _(eb):
                rows = lax.iota(jnp.int32, 16) + eb

                def dot_head(lo):
                    def body(d, a):
                        dv = jnp.full((16,), d, jnp.int32)
                        sc = plsc.load_gather(srows, [rows, dv])
                        dc = plsc.load_gather(drows, [rows, dv])
                        return a + sc * dc
                    return lax.fori_loop(lo, lo + _D, body, zero16,
                                         unroll=8)

                w0 = jnp.exp(dot_head(0) * inv_sqrt_d)
                w1 = jnp.exp(dot_head(_D) * inv_sqrt_d)
                plsc.store_scatter(scaled, [rows, jnp.full((16,), _F, jnp.int32)], w0)
                plsc.store_scatter(scaled, [rows, jnp.full((16,), _F + 1, jnp.int32)], w1)

                def scale_head(lo, w):
                    def body(d, _):
                        dv = jnp.full((16,), d, jnp.int32)
                        sc = plsc.load_gather(srows, [rows, dv])
                        plsc.store_scatter(scaled, [rows, dv], sc * w)
                        return 0
                    lax.fori_loop(lo, lo + _D, body, 0, unroll=8)

                scale_head(0, w0)
                scale_head(_D, w1)

            pltpu.sync_copy(scaled, acc.at[didx], add=True)

        plsc.subcore_barrier()

        pltpu.sync_copy(acc.at[pl.ds(row0, rows_per_tile)],
                        out_hbm.at[pl.ds(tbase + row0, rows_per_tile)])

    return k(h2, src, dst)


def _tc_normalize(acc2, n):
    nb = 2000

    def body(a_ref, o_ref):
        a = a_ref[...]
        d0 = a[:, _F:_F + 1] + 1e-9
        d1 = a[:, _F + 1:_F + 2] + 1e-9
        o_ref[...] = jnp.concatenate(
            [a[:, 0:_D] / d0, a[:, _D:_F] / d1], axis=1)

    return pl.pallas_call(
        body,
        grid=(2 * n // nb,),
        in_specs=[pl.BlockSpec((nb, _ACC_W), lambda i: (i, 0))],
        out_specs=pl.BlockSpec((nb, _F), lambda i: (i, 0)),
        out_shape=jax.ShapeDtypeStruct((2 * n, _F), jnp.float32),
    )(acc2)


def kernel(feat, edge_index, W):
    n = feat.shape[0]
    e = edge_index.shape[1]
    h = _tc_project(feat, W, n)
    h2 = h.reshape(n, 2, _F).transpose(1, 0, 2).reshape(2 * n, _F)
    acc2 = _sc_gat(h2, edge_index[0], edge_index[1], n, e)
    out2 = _tc_normalize(acc2, n)
    return out2.reshape(2, n, 2, _D).transpose(1, 0, 2, 3).reshape(n, _HEADS, _D)


# speedup vs baseline: 26.4284x; 2.0530x over previous
import functools

import jax
import jax.numpy as jnp
from jax import lax
from jax.experimental import pallas as pl
from jax.experimental.pallas import tpu as pltpu
from jax.experimental.pallas import tpu_sc as plsc

_HEADS = 4
_D = 32
_F = 64
_ACC_W = 80


def _tc_project(feat, W, n):
    nb = 2000
    k = feat.shape[1]

    def body(f_ref, w_ref, o_ref):
        o_ref[...] = jnp.dot(f_ref[...], w_ref[...],
                             preferred_element_type=jnp.float32)

    return pl.pallas_call(
        body,
        grid=(n // nb,),
        in_specs=[
            pl.BlockSpec((nb, k), lambda i: (i, 0)),
            pl.BlockSpec((k, 2 * _F), lambda i: (0, 0)),
        ],
        out_specs=pl.BlockSpec((nb, 2 * _F), lambda i: (i, 0)),
        out_shape=jax.ShapeDtypeStruct((n, 2 * _F), jnp.float32),
    )(feat, W)


def _sc_gat(h2, src, dst, n, e):
    num_tiles = 16
    per_tile_e = e // num_tiles
    C = 80
    nc = per_tile_e // C
    rows_per_tile = n // num_tiles
    RB = 125
    row_blocks = rows_per_tile // RB
    inv_sqrt_d = float(1.0 / (_D ** 0.5))
    CB = C * _F
    SB = C * _ACC_W

    mesh = plsc.VectorSubcoreMesh(core_axis_name="c", subcore_axis_name="s")

    @functools.partial(
        pl.kernel,
        out_type=jax.ShapeDtypeStruct((2 * n, _ACC_W), jnp.float32),
        mesh=mesh,
        compiler_params=pltpu.CompilerParams(use_tc_tiling_on_sc=False,
                                             needs_layout_passes=False),
        scratch_types=[
            pltpu.VMEM((2, C), jnp.int32),
            pltpu.VMEM((4, C), jnp.int32),
            pltpu.VMEM((2, C), jnp.int32),
            pltpu.VMEM((2 * C, _F), jnp.float32),
            pltpu.VMEM((2 * C, _F), jnp.float32),
            pltpu.VMEM((3 * C, _ACC_W), jnp.float32),
            pltpu.VMEM((RB, _ACC_W), jnp.float32),
            pltpu.VMEM_SHARED((n, _ACC_W), jnp.float32),
            pltpu.SemaphoreType.DMA,
            pltpu.SemaphoreType.DMA,
            pltpu.SemaphoreType.DMA,
        ],
    )
    def k(h_hbm, src_hbm, dst_hbm, out_hbm,
          sidx, didx, gdidx, srows, drows, scaled, zbuf, acc,
          isem, gsem, ssem):
        cid = lax.axis_index("c")
        sid = lax.axis_index("s")
        zero16 = jnp.zeros((16,), jnp.float32)
        tbase = cid * n
        lane64 = lax.iota(jnp.int32, 16) * _F
        lane80 = lax.iota(jnp.int32, 16) * _ACC_W

        @pl.loop(0, RB)
        def _(r):
            for j in range(_ACC_W // 16):
                zbuf[r, pl.ds(16 * j, 16)] = zero16

        row0 = sid * rows_per_tile
        for b in range(row_blocks):
            pltpu.sync_copy(zbuf, acc.at[pl.ds(row0 + b * RB, RB)])

        @pl.loop(0, 3 * C)
        def _(ei):
            scaled[ei, pl.ds(_F, 16)] = zero16

        plsc.subcore_barrier()

        def fire_idx(kk):
            b = sid * per_tile_e + kk * C
            pltpu.async_copy(src_hbm.at[pl.ds(b, C)],
                             sidx.at[lax.rem(kk, 2)], isem)
            pltpu.async_copy(dst_hbm.at[pl.ds(b, C)],
                             didx.at[lax.rem(kk, 4)], isem)

        def wait_idx():
            pltpu.make_async_copy(src_hbm.at[pl.ds(0, C)], sidx.at[0],
                                  isem).wait()
            pltpu.make_async_copy(dst_hbm.at[pl.ds(0, C)], didx.at[0],
                                  isem).wait()

        def prep_fire_gathers(kk):
            x = lax.rem(kk, 2)
            m4 = lax.rem(kk, 4)

            @pl.loop(0, C, step=16)
            def _(i):
                sidx[x, pl.ds(i, 16)] = sidx[x, pl.ds(i, 16)] + tbase
                gdidx[x, pl.ds(i, 16)] = didx[m4, pl.ds(i, 16)] + tbase

            pltpu.async_copy(h_hbm.at[sidx.at[x]],
                             srows.at[pl.ds(x * C, C)], gsem)
            pltpu.async_copy(h_hbm.at[gdidx.at[x]],
                             drows.at[pl.ds(x * C, C)], gsem)

        def wait_gathers():
            pltpu.make_async_copy(h_hbm.at[sidx.at[0]],
                                  srows.at[pl.ds(0, C)], gsem).wait()
            pltpu.make_async_copy(h_hbm.at[gdidx.at[0]],
                                  drows.at[pl.ds(0, C)], gsem).wait()

        def fire_scatter(kk):
            m3 = lax.rem(kk, 3)
            m4 = lax.rem(kk, 4)
            pltpu.async_copy(scaled.at[pl.ds(m3 * C, C)],
                             acc.at[didx.at[m4]], ssem, add=True)

        def wait_scatter():
            pltpu.make_async_copy(scaled.at[pl.ds(0, C)],
                                  acc.at[didx.at[0]], ssem).wait()

        def compute(kk):
            x = lax.rem(kk, 2)
            m3 = lax.rem(kk, 3)

            @pl.loop(0, C, step=16)
            def _(eb):
                rowv = lax.iota(jnp.int32, 16) + (x * C + eb)
                srowv = lax.iota(jnp.int32, 16) + (m3 * C + eb)

                def head(lo):
                    accs = [zero16, zero16, zero16, zero16]
                    svals = []
                    for d in range(lo, lo + _D):
                        dv = jnp.full((16,), d, jnp.int32)
                        sv = plsc.load_gather(srows, [rowv, dv])
                        tv = plsc.load_gather(drows, [rowv, dv])
                        svals.append(sv)
                        accs[d % 4] = accs[d % 4] + sv * tv
                    ev = (accs[0] + accs[1]) + (accs[2] + accs[3])
                    w = jnp.exp(ev * inv_sqrt_d)
                    for i, d in enumerate(range(lo, lo + _D)):
                        dv = jnp.full((16,), d, jnp.int32)
                        plsc.store_scatter(scaled, [srowv, dv], svals[i] * w)
                    return w

                w0 = head(0)
                w1 = head(_D)
                plsc.store_scatter(scaled, [srowv, jnp.full((16,), _F, jnp.int32)], w0)
                plsc.store_scatter(scaled, [srowv, jnp.full((16,), _F + 1, jnp.int32)], w1)

        fire_idx(0)
        fire_idx(1)
        wait_idx()
        prep_fire_gathers(0)

        @pl.loop(0, nc)
        def _(kk):
            @pl.when(kk >= 1)
            def _():
                fire_scatter(kk - 1)

            @pl.when(kk >= 2)
            def _():
                wait_scatter()

            wait_gathers()

            @pl.when(kk + 2 < nc)
            def _():
                fire_idx(kk + 2)

            @pl.when(kk + 1 < nc)
            def _():
                wait_idx()
                prep_fire_gathers(kk + 1)

            compute(kk)

        fire_scatter(nc - 1)
        wait_scatter()
        wait_scatter()

        plsc.subcore_barrier()

        pltpu.sync_copy(acc.at[pl.ds(row0, rows_per_tile)],
                        out_hbm.at[pl.ds(tbase + row0, rows_per_tile)])

    return k(h2, src, dst)


def _tc_normalize(acc2, n):
    nb = 2000

    def body(a_ref, o_ref):
        a = a_ref[...]
        d0 = a[:, _F:_F + 1] + 1e-9
        d1 = a[:, _F + 1:_F + 2] + 1e-9
        o_ref[...] = jnp.concatenate(
            [a[:, 0:_D] / d0, a[:, _D:_F] / d1], axis=1)

    return pl.pallas_call(
        body,
        grid=(2 * n // nb,),
        in_specs=[pl.BlockSpec((nb, _ACC_W), lambda i: (i, 0))],
        out_specs=pl.BlockSpec((nb, _F), lambda i: (i, 0)),
        out_shape=jax.ShapeDtypeStruct((2 * n, _F), jnp.float32),
    )(acc2)


def kernel(feat, edge_index, W):
    n = feat.shape[0]
    e = edge_index.shape[1]
    h = _tc_project(feat, W, n)
    h2 = h.reshape(n, 2, _F).transpose(1, 0, 2).reshape(2 * n, _F)
    acc2 = _sc_gat(h2, edge_index[0], edge_index[1], n, e)
    out2 = _tc_normalize(acc2, n)
    return out2.reshape(2, n, 2, _D).transpose(1, 0, 2, 3).reshape(n, _HEADS, _D)
